# baseline (speedup 1.0000x reference)
import dataclasses
import functools

import jax
import jax.numpy as jnp
from jax import lax
from jax.experimental import pallas as pl
from jax.experimental.pallas import tpu as pltpu
from jax.experimental.pallas import tpu_sc as plsc

N = 10000
E = 320000
D = 128
H = 128
OUT = 8

NC = 2
NS = 16
TILES = NC * NS
PER_TILE = E // TILES
K = 128
PER_TILE_P = 10240
EP = PER_TILE_P * TILES
CHP = PER_TILE_P // K
NPAD = N + 8
RPS = 624
TAIL = N - RPS * NS

_vec_mesh = plsc.VectorSubcoreMesh(core_axis_name="c", subcore_axis_name="s")


def _striped_copy(s, get_src, get_dst):
    sl = pl.ds(s * RPS, RPS)
    pltpu.sync_copy(get_src(sl), get_dst(sl))

    @pl.when(s == NS - 1)
    def _():
        tl = pl.ds(RPS * NS, TAIL)
        pltpu.sync_copy(get_src(tl), get_dst(tl))


_cp_no_layout = pltpu.CompilerParams()
if "needs_layout_passes" in pltpu.CompilerParams.__dataclass_fields__:
    _cp_no_layout = dataclasses.replace(_cp_no_layout, needs_layout_passes=False)


@functools.partial(
    pl.kernel,
    out_type=jax.ShapeDtypeStruct((TILES, N), jnp.float32),
    mesh=_vec_mesh,
    compiler_params=_cp_no_layout,
    scratch_types=[
        pltpu.VMEM((PER_TILE,), jnp.int32),
        pltpu.VMEM((N,), jnp.float32),
    ],
)
def _deg_kernel(dst_hbm, zeros_hbm, out_hbm, dst_v, deg_v):
    c = lax.axis_index("c")
    s = lax.axis_index("s")
    tid = s * NC + c
    pltpu.sync_copy(dst_hbm.at[tid], dst_v)
    pltpu.sync_copy(zeros_hbm, deg_v)
    ones = jnp.full((16,), 1.0, jnp.float32)

    @pl.loop(0, PER_TILE // 16)
    def _(i):
        idx = dst_v[pl.ds(i * 16, 16)]
        plsc.addupdate_scatter(deg_v, [idx], ones)

    pltpu.sync_copy(deg_v, out_hbm.at[tid])


PAIRS = CHP // 2 - 1


@functools.partial(
    pl.kernel,
    out_type=jax.ShapeDtypeStruct((NC, N, H), jnp.float32),
    mesh=_vec_mesh,
    scratch_types=[
        pltpu.VMEM((K,), jnp.int32),
        pltpu.VMEM((K,), jnp.int32),
        pltpu.VMEM((CHP, K), jnp.int32),
        pltpu.VMEM((K, H), jnp.float32),
        pltpu.VMEM((K, H), jnp.float32),
        pltpu.VMEM_SHARED((NPAD, H), jnp.float32),
        pltpu.SemaphoreType.DMA,
        pltpu.SemaphoreType.DMA,
        pltpu.SemaphoreType.DMA,
        pltpu.SemaphoreType.DMA,
    ],
)
def _edge_kernel(hp_hbm, src_hbm, dst_hbm, out_hbm,
                 idx_a, idx_b, dst_v, rows_a, rows_b, acc,
                 si_a, si_b, sg_a, sg_b):
    c = lax.axis_index("c")
    s = lax.axis_index("s")
    tid = s * NC + c
    pltpu.sync_copy(dst_hbm.at[tid], dst_v)
    _striped_copy(s, lambda d: hp_hbm.at[d], lambda d: acc.at[d])
    plsc.subcore_barrier()

    def idx_start(j, buf, sem):
        pltpu.async_copy(src_hbm.at[tid, pl.ds(j * K, K)], buf, sem)

    def idx_wait(j, buf, sem):
        pltpu.make_async_copy(src_hbm.at[tid, pl.ds(j * K, K)], buf, sem).wait()

    def gather_start(buf_idx, buf, sem):
        pltpu.async_copy(hp_hbm.at[buf_idx], buf, sem)

    def gather_wait(buf_idx, buf, sem):
        pltpu.make_async_copy(hp_hbm.at[buf_idx], buf, sem).wait()

    def scatter(j, buf):
        pltpu.sync_copy(buf, acc.at[dst_v.at[j]], add=True)

    pltpu.sync_copy(src_hbm.at[tid, pl.ds(0, K)], idx_a)
    gather_start(idx_a, rows_a, sg_a)
    idx_start(1, idx_b, si_b)

    @pl.loop(0, PAIRS)
    def _(i):
        j = 2 * i
        gather_wait(idx_a, rows_a, sg_a)
        idx_wait(j + 1, idx_b, si_b)
        gather_start(idx_b, rows_b, sg_b)
        idx_start(j + 2, idx_a, si_a)
        scatter(j, rows_a)
        gather_wait(idx_b, rows_b, sg_b)
        idx_wait(j + 2, idx_a, si_a)
        gather_start(idx_a, rows_a, sg_a)
        idx_start(j + 3, idx_b, si_b)
        scatter(j + 1, rows_b)

    gather_wait(idx_a, rows_a, sg_a)
    idx_wait(CHP - 1, idx_b, si_b)
    gather_start(idx_b, rows_b, sg_b)
    scatter(CHP - 2, rows_a)
    gather_wait(idx_b, rows_b, sg_b)
    scatter(CHP - 1, rows_b)

    plsc.subcore_barrier()
    _striped_copy(s, lambda d: acc.at[d], lambda d: out_hbm.at[c, d])


BR = 1000
GB = N // BR


def _mm(a, b):
    return jnp.dot(a, b, precision=lax.Precision.DEFAULT,
                   preferred_element_type=jnp.float32)


def _ln(t, g, b, eps=1e-5):
    mu = jnp.mean(t, axis=-1, keepdims=True)
    var = jnp.mean((t - mu) ** 2, axis=-1, keepdims=True)
    return (t - mu) * lax.rsqrt(var + eps) * g + b


def _tc_pre_body(degp, x, w1, dinv_o, hp_o):
    deg = jnp.sum(degp[0], axis=-1)[:, None] + 1.0
    dinv = lax.rsqrt(jnp.maximum(deg, 1.0))
    dinv_o[...] = dinv
    hp_o[...] = _mm(x[...], w1[...]) * dinv


_tc_pre = pl.pallas_call(
    _tc_pre_body,
    grid=(GB,),
    in_specs=[
        pl.BlockSpec((1, BR, TILES), lambda i: (i, 0, 0)),
        pl.BlockSpec((BR, D), lambda i: (i, 0)),
        pl.BlockSpec((D, H), lambda i: (0, 0)),
    ],
    out_specs=[
        pl.BlockSpec((BR, 1), lambda i: (i, 0)),
        pl.BlockSpec((BR, H), lambda i: (i, 0)),
    ],
    out_shape=[
        jax.ShapeDtypeStruct((N, 1), jnp.float32),
        jax.ShapeDtypeStruct((N, H), jnp.float32),
    ],
)


def _tc_mid_body(p, hp, dinv, b1, g1, bb1, w2, x1_o, h2p_o):
    dv = dinv[...]
    t = (p[0] + p[1] - hp[...]) * dv + b1[...]
    t = jnp.maximum(_ln(t, g1[...], bb1[...]), 0.0)
    x1_o[...] = t
    h2p_o[...] = _mm(t, w2[...]) * dv


_tc_mid = pl.pallas_call(
    _tc_mid_body,
    grid=(GB,),
    in_specs=[
        pl.BlockSpec((NC, BR, H), lambda i: (0, i, 0)),
        pl.BlockSpec((BR, H), lambda i: (i, 0)),
        pl.BlockSpec((BR, 1), lambda i: (i, 0)),
        pl.BlockSpec((1, H), lambda i: (0, 0)),
        pl.BlockSpec((1, H), lambda i: (0, 0)),
        pl.BlockSpec((1, H), lambda i: (0, 0)),
        pl.BlockSpec((H, H), lambda i: (0, 0)),
    ],
    out_specs=[
        pl.BlockSpec((BR, H), lambda i: (i, 0)),
        pl.BlockSpec((BR, H), lambda i: (i, 0)),
    ],
    out_shape=[
        jax.ShapeDtypeStruct((N, H), jnp.float32),
        jax.ShapeDtypeStruct((N, H), jnp.float32),
    ],
)


def _tc_head_body(q, hp, dinv, b2, g2, bb2, x1,
                  wa1, ba1, ga, bba, wa2, ba2,
                  wc1, bc1, gc, bbc, wc2, bc2,
                  probs_o, vals_o):
    dv = dinv[...]
    t = (q[0] + q[1] - hp[...]) * dv + b2[...]
    x2 = jnp.maximum(_ln(t, g2[...], bb2[...]), 0.0)
    xs = x2 + x1[...]

    a = jnp.maximum(_mm(xs, wa1[...]) + ba1[...], 0.0)
    a = _ln(a, ga[...], bba[...])
    logits = _mm(a, wa2[...]) + ba2[...]
    m = jnp.max(logits, axis=-1, keepdims=True)
    e = jnp.exp(logits - m)
    probs_o[...] = e / jnp.sum(e, axis=-1, keepdims=True)

    cch = jnp.maximum(_mm(xs, wc1[...]) + bc1[...], 0.0)
    cch = _ln(cch, gc[...], bbc[...])
    vals_o[...] = _mm(cch, wc2[...]) + bc2[...]


def _full(shape):
    return pl.BlockSpec(shape, lambda *_: tuple(0 for _ in shape))


_tc_head = pl.pallas_call(
    _tc_head_body,
    grid=(GB,),
    in_specs=[
        pl.BlockSpec((NC, BR, H), lambda i: (0, i, 0)),
        pl.BlockSpec((BR, H), lambda i: (i, 0)),
        pl.BlockSpec((BR, 1), lambda i: (i, 0)),
        _full((1, H)), _full((1, H)), _full((1, H)),
        pl.BlockSpec((BR, H), lambda i: (i, 0)),
        _full((H, H)), _full((1, H)), _full((1, H)), _full((1, H)),
        _full((H, OUT)), _full((1, OUT)),
        _full((H, H)), _full((1, H)), _full((1, H)), _full((1, H)),
        _full((H, 1)), _full((1, 1)),
    ],
    out_specs=[
        pl.BlockSpec((BR, OUT), lambda i: (i, 0)),
        pl.BlockSpec((BR, 1), lambda i: (i, 0)),
    ],
    out_shape=[
        jax.ShapeDtypeStruct((N, OUT), jnp.float32),
        jax.ShapeDtypeStruct((N, 1), jnp.float32),
    ],
)


def kernel(x, edge_index, W1, b1, ln1_g, ln1_b, W2, b2, ln2_g, ln2_b,
           Wa1, ba1, lna_g, lna_b, Wa2, ba2, Wc1, bc1, lnc_g, lnc_b, Wc2, bc2):
    pad = jnp.concatenate(
        [jnp.zeros((1, EP - E), jnp.int32),
         jnp.full((1, EP - E), N, jnp.int32)], axis=0)
    ei_p = jnp.concatenate([edge_index, pad], axis=1)
    src_r = ei_p[0].reshape(TILES, PER_TILE_P)
    dst_r = ei_p[1].reshape(TILES, CHP, K)
    dst_flat = edge_index[1].reshape(TILES, PER_TILE)
    zeros_n = jnp.zeros((N,), jnp.float32)

    def r(v):
        return v.reshape(1, -1)

    degp = _deg_kernel(dst_flat, zeros_n)
    degp_t = degp.T.reshape(GB, BR, TILES)
    dinv, h1p = _tc_pre(degp_t, x, W1)
    p = _edge_kernel(h1p, src_r, dst_r)
    x1, h2p = _tc_mid(p, h1p, dinv, r(b1), r(ln1_g), r(ln1_b), W2)
    q = _edge_kernel(h2p, src_r, dst_r)
    probs, vals = _tc_head(q, h2p, dinv, r(b2), r(ln2_g), r(ln2_b), x1,
                           Wa1, r(ba1), r(lna_g), r(lna_b), Wa2, r(ba2),
                           Wc1, r(bc1), r(lnc_g), r(lnc_b), Wc2, r(bc2))
    return probs, vals


# speedup vs baseline: 3.1805x; 3.1805x over previous
import dataclasses
import functools

import jax
import jax.numpy as jnp
from jax import lax
from jax.experimental import pallas as pl
from jax.experimental.pallas import tpu as pltpu
from jax.experimental.pallas import tpu_sc as plsc

N = 10000
E = 320000
D = 128
H = 128
OUT = 8

NC = 2
NS = 16
TILES = NC * NS
PER_TILE = E // TILES
K = 80
CH = PER_TILE // K
RPS = 624
TAIL = N - RPS * NS

_vec_mesh = plsc.VectorSubcoreMesh(core_axis_name="c", subcore_axis_name="s")


def _striped_copy(s, get_src, get_dst):
    sl = pl.ds(s * RPS, RPS)
    pltpu.sync_copy(get_src(sl), get_dst(sl))

    @pl.when(s == NS - 1)
    def _():
        tl = pl.ds(RPS * NS, TAIL)
        pltpu.sync_copy(get_src(tl), get_dst(tl))


_cp_no_layout = pltpu.CompilerParams()
if "needs_layout_passes" in pltpu.CompilerParams.__dataclass_fields__:
    _cp_no_layout = dataclasses.replace(_cp_no_layout, needs_layout_passes=False)


@functools.partial(
    pl.kernel,
    out_type=jax.ShapeDtypeStruct((TILES, N), jnp.float32),
    mesh=_vec_mesh,
    compiler_params=_cp_no_layout,
    scratch_types=[
        pltpu.VMEM((CH, K), jnp.int32),
        pltpu.VMEM((N,), jnp.float32),
    ],
)
def _deg_kernel(dst_hbm, zeros_hbm, out_hbm, dst_v, deg_v):
    c = lax.axis_index("c")
    s = lax.axis_index("s")
    tid = s * NC + c
    pltpu.sync_copy(dst_hbm.at[tid], dst_v)
    pltpu.sync_copy(zeros_hbm, deg_v)
    ones = jnp.full((16,), 1.0, jnp.float32)

    @pl.loop(0, CH)
    def _(j):
        for l in range(K // 16):
            idx = dst_v[j, pl.ds(l * 16, 16)]
            plsc.addupdate_scatter(deg_v, [idx], ones)

    pltpu.sync_copy(deg_v, out_hbm.at[tid])


PAIRS = (CH - 1) // 2


@functools.partial(
    pl.kernel,
    out_type=jax.ShapeDtypeStruct((NC, N, H), jnp.float32),
    mesh=_vec_mesh,
    scratch_types=[
        pltpu.VMEM((PER_TILE,), jnp.int32),
        pltpu.VMEM((CH, K), jnp.int32),
        pltpu.VMEM((K, H), jnp.float32),
        pltpu.VMEM((K, H), jnp.float32),
        pltpu.VMEM_SHARED((N, H), jnp.float32),
        pltpu.SemaphoreType.DMA,
        pltpu.SemaphoreType.DMA,
    ],
)
def _edge_kernel(hp_hbm, src_hbm, dst_hbm, out_hbm,
                 src_v, dst_v, rows_a, rows_b, acc, sg_a, sg_b):
    c = lax.axis_index("c")
    s = lax.axis_index("s")
    tid = s * NC + c
    pltpu.sync_copy(src_hbm.at[tid], src_v)
    pltpu.sync_copy(dst_hbm.at[tid], dst_v)
    _striped_copy(s, lambda d: hp_hbm.at[d], lambda d: acc.at[d])
    plsc.subcore_barrier()

    def gather_start(j, buf, sem):
        pltpu.async_copy(hp_hbm.at[src_v.at[pl.ds(j * K, K)]], buf, sem)

    def gather_wait(j, buf, sem):
        pltpu.make_async_copy(
            hp_hbm.at[src_v.at[pl.ds(j * K, K)]], buf, sem).wait()

    def scatter(j, buf):
        pltpu.sync_copy(buf, acc.at[dst_v.at[j]], add=True)

    gather_start(0, rows_a, sg_a)

    @pl.loop(0, PAIRS)
    def _(i):
        j = 2 * i
        gather_wait(j, rows_a, sg_a)
        gather_start(j + 1, rows_b, sg_b)
        scatter(j, rows_a)
        gather_start(j + 2, rows_a, sg_a)
        gather_wait(j + 1, rows_b, sg_b)
        scatter(j + 1, rows_b)

    gather_wait(CH - 1, rows_a, sg_a)
    scatter(CH - 1, rows_a)

    plsc.subcore_barrier()
    _striped_copy(s, lambda d: acc.at[d], lambda d: out_hbm.at[c, d])


BR = 1000
GB = N // BR


def _mm(a, b):
    return jnp.dot(a, b, precision=lax.Precision.DEFAULT,
                   preferred_element_type=jnp.float32)


def _ln(t, g, b, eps=1e-5):
    mu = jnp.mean(t, axis=-1, keepdims=True)
    var = jnp.mean((t - mu) ** 2, axis=-1, keepdims=True)
    return (t - mu) * lax.rsqrt(var + eps) * g + b


def _tc_mm1_body(x, w1, h1_o):
    h1_o[...] = _mm(x[...], w1[...])


_tc_mm1 = pl.pallas_call(
    _tc_mm1_body,
    grid=(GB,),
    in_specs=[
        pl.BlockSpec((BR, D), lambda i: (i, 0)),
        pl.BlockSpec((D, H), lambda i: (0, 0)),
    ],
    out_specs=pl.BlockSpec((BR, H), lambda i: (i, 0)),
    out_shape=jax.ShapeDtypeStruct((N, H), jnp.float32),
)


def _tc_pre_body(degp, h1, dinv_o, hp_o):
    deg = jnp.sum(degp[0], axis=-1)[:, None] + 1.0
    dinv = lax.rsqrt(jnp.maximum(deg, 1.0))
    dinv_o[...] = dinv
    hp_o[...] = h1[...] * dinv


_tc_pre = pl.pallas_call(
    _tc_pre_body,
    grid=(GB,),
    in_specs=[
        pl.BlockSpec((1, BR, TILES), lambda i: (i, 0, 0)),
        pl.BlockSpec((BR, H), lambda i: (i, 0)),
    ],
    out_specs=[
        pl.BlockSpec((BR, 1), lambda i: (i, 0)),
        pl.BlockSpec((BR, H), lambda i: (i, 0)),
    ],
    out_shape=[
        jax.ShapeDtypeStruct((N, 1), jnp.float32),
        jax.ShapeDtypeStruct((N, H), jnp.float32),
    ],
)


def _tc_mid_body(p, hp, dinv, b1, g1, bb1, w2, x1_o, h2p_o):
    dv = dinv[...]
    t = (p[0] + p[1] - hp[...]) * dv + b1[...]
    t = jnp.maximum(_ln(t, g1[...], bb1[...]), 0.0)
    x1_o[...] = t
    h2p_o[...] = _mm(t, w2[...]) * dv


_tc_mid = pl.pallas_call(
    _tc_mid_body,
    grid=(GB,),
    in_specs=[
        pl.BlockSpec((NC, BR, H), lambda i: (0, i, 0)),
        pl.BlockSpec((BR, H), lambda i: (i, 0)),
        pl.BlockSpec((BR, 1), lambda i: (i, 0)),
        pl.BlockSpec((1, H), lambda i: (0, 0)),
        pl.BlockSpec((1, H), lambda i: (0, 0)),
        pl.BlockSpec((1, H), lambda i: (0, 0)),
        pl.BlockSpec((H, H), lambda i: (0, 0)),
    ],
    out_specs=[
        pl.BlockSpec((BR, H), lambda i: (i, 0)),
        pl.BlockSpec((BR, H), lambda i: (i, 0)),
    ],
    out_shape=[
        jax.ShapeDtypeStruct((N, H), jnp.float32),
        jax.ShapeDtypeStruct((N, H), jnp.float32),
    ],
)


def _tc_head_body(q, hp, dinv, b2, g2, bb2, x1,
                  wa1, ba1, ga, bba, wa2, ba2,
                  wc1, bc1, gc, bbc, wc2, bc2,
                  probs_o, vals_o):
    dv = dinv[...]
    t = (q[0] + q[1] - hp[...]) * dv + b2[...]
    x2 = jnp.maximum(_ln(t, g2[...], bb2[...]), 0.0)
    xs = x2 + x1[...]

    a = jnp.maximum(_mm(xs, wa1[...]) + ba1[...], 0.0)
    a = _ln(a, ga[...], bba[...])
    logits = _mm(a, wa2[...]) + ba2[...]
    m = jnp.max(logits, axis=-1, keepdims=True)
    e = jnp.exp(logits - m)
    probs_o[...] = e / jnp.sum(e, axis=-1, keepdims=True)

    cch = jnp.maximum(_mm(xs, wc1[...]) + bc1[...], 0.0)
    cch = _ln(cch, gc[...], bbc[...])
    vals_o[...] = _mm(cch, wc2[...]) + bc2[...]


def _full(shape):
    return pl.BlockSpec(shape, lambda *_: tuple(0 for _ in shape))


_tc_head = pl.pallas_call(
    _tc_head_body,
    grid=(GB,),
    in_specs=[
        pl.BlockSpec((NC, BR, H), lambda i: (0, i, 0)),
        pl.BlockSpec((BR, H), lambda i: (i, 0)),
        pl.BlockSpec((BR, 1), lambda i: (i, 0)),
        _full((1, H)), _full((1, H)), _full((1, H)),
        pl.BlockSpec((BR, H), lambda i: (i, 0)),
        _full((H, H)), _full((1, H)), _full((1, H)), _full((1, H)),
        _full((H, OUT)), _full((1, OUT)),
        _full((H, H)), _full((1, H)), _full((1, H)), _full((1, H)),
        _full((H, 1)), _full((1, 1)),
    ],
    out_specs=[
        pl.BlockSpec((BR, OUT), lambda i: (i, 0)),
        pl.BlockSpec((BR, 1), lambda i: (i, 0)),
    ],
    out_shape=[
        jax.ShapeDtypeStruct((N, OUT), jnp.float32),
        jax.ShapeDtypeStruct((N, 1), jnp.float32),
    ],
)


def kernel(x, edge_index, W1, b1, ln1_g, ln1_b, W2, b2, ln2_g, ln2_b,
           Wa1, ba1, lna_g, lna_b, Wa2, ba2, Wc1, bc1, lnc_g, lnc_b, Wc2, bc2):
    src_r = edge_index[0].reshape(TILES, PER_TILE)
    dst_r = edge_index[1].reshape(TILES, CH, K)
    zeros_n = jnp.zeros((N,), jnp.float32)

    def r(v):
        return v.reshape(1, -1)

    h1 = _tc_mm1(x, W1)
    degp = _deg_kernel(dst_r, zeros_n)
    degp_t = degp.T.reshape(GB, BR, TILES)
    dinv, h1p = _tc_pre(degp_t, h1)
    p = _edge_kernel(h1p, src_r, dst_r)
    x1, h2p = _tc_mid(p, h1p, dinv, r(b1), r(ln1_g), r(ln1_b), W2)
    q = _edge_kernel(h2p, src_r, dst_r)
    probs, vals = _tc_head(q, h2p, dinv, r(b2), r(ln2_g), r(ln2_b), x1,
                           Wa1, r(ba1), r(lna_g), r(lna_b), Wa2, r(ba2),
                           Wc1, r(bc1), r(lnc_g), r(lnc_b), Wc2, r(bc2))
    return probs, vals


# speedup vs baseline: 3.2781x; 1.0307x over previous
import dataclasses
import functools

import jax
import jax.numpy as jnp
from jax import lax
from jax.experimental import pallas as pl
from jax.experimental.pallas import tpu as pltpu
from jax.experimental.pallas import tpu_sc as plsc

N = 10000
E = 320000
D = 128
H = 128
OUT = 8

NC = 2
NS = 16
TILES = NC * NS
PER_TILE = E // TILES
K = 80
CH = PER_TILE // K
RPS = 624
TAIL = N - RPS * NS

_vec_mesh = plsc.VectorSubcoreMesh(core_axis_name="c", subcore_axis_name="s")


def _striped_copy(s, get_src, get_dst):
    sl = pl.ds(s * RPS, RPS)
    pltpu.sync_copy(get_src(sl), get_dst(sl))

    @pl.when(s == NS - 1)
    def _():
        tl = pl.ds(RPS * NS, TAIL)
        pltpu.sync_copy(get_src(tl), get_dst(tl))


_cp_no_layout = pltpu.CompilerParams()
if "needs_layout_passes" in pltpu.CompilerParams.__dataclass_fields__:
    _cp_no_layout = dataclasses.replace(_cp_no_layout, needs_layout_passes=False)


@functools.partial(
    pl.kernel,
    out_type=jax.ShapeDtypeStruct((TILES, N), jnp.float32),
    mesh=_vec_mesh,
    compiler_params=_cp_no_layout,
    scratch_types=[
        pltpu.VMEM((CH, K), jnp.int32),
        pltpu.VMEM((N,), jnp.float32),
    ],
)
def _deg_kernel(dst_hbm, zeros_hbm, out_hbm, dst_v, deg_v):
    c = lax.axis_index("c")
    s = lax.axis_index("s")
    tid = s * NC + c
    pltpu.sync_copy(dst_hbm.at[tid], dst_v)
    pltpu.sync_copy(zeros_hbm, deg_v)
    ones = jnp.full((16,), 1.0, jnp.float32)

    @pl.loop(0, CH)
    def _(j):
        for l in range(K // 16):
            idx = dst_v[j, pl.ds(l * 16, 16)]
            plsc.addupdate_scatter(deg_v, [idx], ones)

    pltpu.sync_copy(deg_v, out_hbm.at[tid])


PAIRS = (CH - 1) // 2


@functools.partial(
    pl.kernel,
    out_type=jax.ShapeDtypeStruct((NC, N, H), jnp.float32),
    mesh=_vec_mesh,
    scratch_types=[
        pltpu.VMEM((PER_TILE,), jnp.int32),
        pltpu.VMEM((CH, K), jnp.int32),
        pltpu.VMEM((K, H), jnp.float32),
        pltpu.VMEM((K, H), jnp.float32),
        pltpu.VMEM_SHARED((N, H), jnp.float32),
        pltpu.SemaphoreType.DMA,
        pltpu.SemaphoreType.DMA,
    ],
)
def _edge_kernel(hp_hbm, src_hbm, dst_hbm, out_hbm,
                 src_v, dst_v, rows_a, rows_b, acc, sg_a, sg_b):
    c = lax.axis_index("c")
    s = lax.axis_index("s")
    tid = s * NC + c
    pltpu.sync_copy(src_hbm.at[tid], src_v)
    pltpu.sync_copy(dst_hbm.at[tid], dst_v)
    _striped_copy(s, lambda d: hp_hbm.at[d], lambda d: acc.at[d])
    plsc.subcore_barrier()

    def gather_start(j, buf, sem):
        pltpu.async_copy(hp_hbm.at[src_v.at[pl.ds(j * K, K)]], buf, sem)

    def gather_wait(j, buf, sem):
        pltpu.make_async_copy(
            hp_hbm.at[src_v.at[pl.ds(j * K, K)]], buf, sem).wait()

    def scatter(j, buf):
        pltpu.sync_copy(buf, acc.at[dst_v.at[j]], add=True)

    gather_start(0, rows_a, sg_a)

    @pl.loop(0, PAIRS)
    def _(i):
        j = 2 * i
        gather_wait(j, rows_a, sg_a)
        gather_start(j + 1, rows_b, sg_b)
        scatter(j, rows_a)
        gather_start(j + 2, rows_a, sg_a)
        gather_wait(j + 1, rows_b, sg_b)
        scatter(j + 1, rows_b)

    gather_wait(CH - 1, rows_a, sg_a)
    scatter(CH - 1, rows_a)

    plsc.subcore_barrier()
    _striped_copy(s, lambda d: acc.at[d], lambda d: out_hbm.at[c, d])


BR = 2000
GB = N // BR


def _mm(a, b):
    return jnp.dot(a, b, precision=lax.Precision.DEFAULT,
                   preferred_element_type=jnp.float32)


def _ln(t, g, b, eps=1e-5):
    mu = jnp.mean(t, axis=-1, keepdims=True)
    var = jnp.mean((t - mu) ** 2, axis=-1, keepdims=True)
    return (t - mu) * lax.rsqrt(var + eps) * g + b


def _tc_mm1_body(x, w1, h1_o):
    h1_o[...] = _mm(x[...], w1[...])


_tc_mm1 = pl.pallas_call(
    _tc_mm1_body,
    grid=(GB,),
    in_specs=[
        pl.BlockSpec((BR, D), lambda i: (i, 0)),
        pl.BlockSpec((D, H), lambda i: (0, 0)),
    ],
    out_specs=pl.BlockSpec((BR, H), lambda i: (i, 0)),
    out_shape=jax.ShapeDtypeStruct((N, H), jnp.float32),
)


def _tc_pre_body(degp, h1, dinv_o, hp_o):
    deg = jnp.sum(degp[0], axis=-1)[:, None] + 1.0
    dinv = lax.rsqrt(jnp.maximum(deg, 1.0))
    dinv_o[...] = dinv
    hp_o[...] = h1[...] * dinv


_tc_pre = pl.pallas_call(
    _tc_pre_body,
    grid=(GB,),
    in_specs=[
        pl.BlockSpec((1, BR, TILES), lambda i: (i, 0, 0)),
        pl.BlockSpec((BR, H), lambda i: (i, 0)),
    ],
    out_specs=[
        pl.BlockSpec((BR, 1), lambda i: (i, 0)),
        pl.BlockSpec((BR, H), lambda i: (i, 0)),
    ],
    out_shape=[
        jax.ShapeDtypeStruct((N, 1), jnp.float32),
        jax.ShapeDtypeStruct((N, H), jnp.float32),
    ],
)


def _tc_mid_body(p, hp, dinv, b1, g1, bb1, w2, x1_o, h2p_o):
    dv = dinv[...]
    t = (p[0] + p[1] - hp[...]) * dv + b1[...]
    t = jnp.maximum(_ln(t, g1[...], bb1[...]), 0.0)
    x1_o[...] = t
    h2p_o[...] = _mm(t, w2[...]) * dv


_tc_mid = pl.pallas_call(
    _tc_mid_body,
    grid=(GB,),
    in_specs=[
        pl.BlockSpec((NC, BR, H), lambda i: (0, i, 0)),
        pl.BlockSpec((BR, H), lambda i: (i, 0)),
        pl.BlockSpec((BR, 1), lambda i: (i, 0)),
        pl.BlockSpec((1, H), lambda i: (0, 0)),
        pl.BlockSpec((1, H), lambda i: (0, 0)),
        pl.BlockSpec((1, H), lambda i: (0, 0)),
        pl.BlockSpec((H, H), lambda i: (0, 0)),
    ],
    out_specs=[
        pl.BlockSpec((BR, H), lambda i: (i, 0)),
        pl.BlockSpec((BR, H), lambda i: (i, 0)),
    ],
    out_shape=[
        jax.ShapeDtypeStruct((N, H), jnp.float32),
        jax.ShapeDtypeStruct((N, H), jnp.float32),
    ],
)


def _tc_head_body(q, hp, dinv, b2, g2, bb2, x1,
                  wa1, ba1, ga, bba, wa2, ba2,
                  wc1, bc1, gc, bbc, wc2, bc2,
                  probs_o, vals_o):
    dv = dinv[...]
    t = (q[0] + q[1] - hp[...]) * dv + b2[...]
    x2 = jnp.maximum(_ln(t, g2[...], bb2[...]), 0.0)
    xs = x2 + x1[...]

    a = jnp.maximum(_mm(xs, wa1[...]) + ba1[...], 0.0)
    a = _ln(a, ga[...], bba[...])
    logits = _mm(a, wa2[...]) + ba2[...]
    m = jnp.max(logits, axis=-1, keepdims=True)
    e = jnp.exp(logits - m)
    probs_o[...] = e / jnp.sum(e, axis=-1, keepdims=True)

    cch = jnp.maximum(_mm(xs, wc1[...]) + bc1[...], 0.0)
    cch = _ln(cch, gc[...], bbc[...])
    vals_o[...] = _mm(cch, wc2[...]) + bc2[...]


def _full(shape):
    return pl.BlockSpec(shape, lambda *_: tuple(0 for _ in shape))


_tc_head = pl.pallas_call(
    _tc_head_body,
    grid=(GB,),
    in_specs=[
        pl.BlockSpec((NC, BR, H), lambda i: (0, i, 0)),
        pl.BlockSpec((BR, H), lambda i: (i, 0)),
        pl.BlockSpec((BR, 1), lambda i: (i, 0)),
        _full((1, H)), _full((1, H)), _full((1, H)),
        pl.BlockSpec((BR, H), lambda i: (i, 0)),
        _full((H, H)), _full((1, H)), _full((1, H)), _full((1, H)),
        _full((H, OUT)), _full((1, OUT)),
        _full((H, H)), _full((1, H)), _full((1, H)), _full((1, H)),
        _full((H, 1)), _full((1, 1)),
    ],
    out_specs=[
        pl.BlockSpec((BR, OUT), lambda i: (i, 0)),
        pl.BlockSpec((BR, 1), lambda i: (i, 0)),
    ],
    out_shape=[
        jax.ShapeDtypeStruct((N, OUT), jnp.float32),
        jax.ShapeDtypeStruct((N, 1), jnp.float32),
    ],
)


def kernel(x, edge_index, W1, b1, ln1_g, ln1_b, W2, b2, ln2_g, ln2_b,
           Wa1, ba1, lna_g, lna_b, Wa2, ba2, Wc1, bc1, lnc_g, lnc_b, Wc2, bc2):
    src_r = edge_index[0].reshape(TILES, PER_TILE)
    dst_r = edge_index[1].reshape(TILES, CH, K)
    zeros_n = jnp.zeros((N,), jnp.float32)

    def r(v):
        return v.reshape(1, -1)

    h1 = _tc_mm1(x, W1)
    degp = _deg_kernel(dst_r, zeros_n)
    degp_t = degp.T.reshape(GB, BR, TILES)
    dinv, h1p = _tc_pre(degp_t, h1)
    p = _edge_kernel(h1p, src_r, dst_r)
    x1, h2p = _tc_mid(p, h1p, dinv, r(b1), r(ln1_g), r(ln1_b), W2)
    q = _edge_kernel(h2p, src_r, dst_r)
    probs, vals = _tc_head(q, h2p, dinv, r(b2), r(ln2_g), r(ln2_b), x1,
                           Wa1, r(ba1), r(lna_g), r(lna_b), Wa2, r(ba2),
                           Wc1, r(bc1), r(lnc_g), r(lnc_b), Wc2, r(bc2))
    return probs, vals
